# baseline (speedup 1.0000x reference)
import jax
import jax.numpy as jnp
from jax import lax
from jax.experimental import pallas as pl
from jax.experimental.pallas import tpu as pltpu
from jax.experimental.pallas import tpu_sc as plsc

NC, NS, L = 2, 16, 16
B, T, D = 4, 8192, 1024
K = T // 8
R = B * T
WPR = 8
CW = T // WPR
NV = CW // L
CHG = 48
MAXCH = (CW + CHG - 1) // CHG + 1
BT = 2048


def _scalar(x):
    if getattr(x, "ndim", 0) == 0:
        return x
    return lax.reduce_max(x, (0,))


def _route_body(h_hbm, sc_hbm, p_hbm,
                sbuf, ukeys, hist, merged, idx2d, gbuf, shared, sem_g, sem_s):
    c = lax.axis_index("c")
    s = lax.axis_index("s")
    b = 2 * c + s // WPR
    jw = s % WPR
    tbase = b * T + jw * CW
    s0 = (s // WPR) * WPR
    lane = lax.iota(jnp.int32, L)

    pltpu.sync_copy(sc_hbm.at[pl.ds(tbase, CW)], sbuf)
    for j in range(NV):
        v = sbuf[pl.ds(L * j, L)]
        xi = lax.bitcast_convert_type(v, jnp.int32)
        sign = xi >> 31
        u = lax.bitcast_convert_type(xi, jnp.uint32) ^ (
            lax.bitcast_convert_type(sign, jnp.uint32) | jnp.uint32(0x80000000))
        ukeys[j] = u

    prefix = jnp.uint32(0)
    krem = jnp.int32(K)
    for p in range(4):
        shift = 24 - 8 * p
        for g in range(256 // L):
            hist[pl.ds(L * g, L)] = jnp.zeros((L,), jnp.int32)

        def hbody(j, _, _p=p, _shift=shift, _prefix=prefix):
            u = ukeys[j]
            binv = lax.bitcast_convert_type(
                (u >> jnp.uint32(_shift)) & jnp.uint32(0xFF), jnp.int32)
            if _p == 0:
                elig = jnp.full((L,), True)
            else:
                elig = (u >> jnp.uint32(_shift + 8)) == _prefix
            counts, lastm = plsc.scan_count(binv, elig)
            plsc.addupdate_scatter(hist, [binv], counts, mask=lastm)
            return 0

        lax.fori_loop(0, NV, hbody, 0)

        pltpu.sync_copy(hist, shared.at[s])
        plsc.subcore_barrier()
        pltpu.sync_copy(shared.at[pl.ds(s0, WPR)], merged)
        plsc.subcore_barrier()

        rem = krem
        kin = jnp.int32(1)
        hsel = jnp.zeros((L,), jnp.int32)
        for g in reversed(range(256 // L)):
            hv = jnp.zeros((L,), jnp.int32)
            for w in range(WPR):
                hv = hv + merged[w, pl.ds(L * g, L)]
            tg = lax.reduce_sum(hv, (0,))
            hit = (rem > 0) & (rem <= tg)
            hitv = jnp.full((L,), hit)
            gsel = jnp.where(hit, jnp.int32(g), jnp.int32(0))
            if g == 256 // L - 1:
                g_star = gsel
            else:
                g_star = jnp.where(hit, gsel, g_star)
            kin = jnp.where(hit, rem, kin)
            hsel = jnp.where(hitv, hv, hsel)
            rem = jnp.where(rem > 0, rem - tg, rem)

        rv = lax.rev(hsel, (0,))
        cum = plsc.cumsum(rv)
        fmask = cum >= kin
        f = _scalar(plsc.all_reduce_ffs(fmask))
        cum_f = lax.reduce_sum(jnp.where(lane == f, cum, 0), (0,))
        rv_f = lax.reduce_sum(jnp.where(lane == f, rv, 0), (0,))
        bin_star = (L - 1) - f
        krem = kin - (cum_f - rv_f)
        prefix = (prefix << jnp.uint32(8)) | jnp.uint32(
            g_star * L + bin_star)

    thresh = prefix

    def sbody(j, off):
        u = ukeys[j]
        m = u >= thresh
        mi = jnp.where(m, jnp.int32(1), jnp.int32(0))
        pos = plsc.cumsum(mi) + off - 1
        tok = lane + (tbase + L * j)
        plsc.store_scatter(idx2d, [pos // CHG, pos % CHG], tok, mask=m)
        return off + _scalar(plsc.all_reduce_population_count(m))

    cnt = lax.fori_loop(0, NV, sbody, jnp.int32(0))

    padn = (CHG - lax.rem(cnt, CHG)) % CHG
    for g in range((CHG + L - 1) // L):
        posv = cnt + (g * L) + lane
        m = (g * L + lane) < padn
        plsc.store_scatter(
            idx2d, [posv // CHG, posv % CHG],
            jnp.full((L,), tbase, jnp.int32), mask=m)
    n_ch = (cnt + padn) // CHG

    def _wait_one_scatter():
        pltpu.make_async_copy(
            gbuf.at[0], p_hbm.at[idx2d.at[0]], sem_s).wait()

    def cbody(j, _):
        slot = lax.rem(j, 2)

        @pl.when(j >= 2)
        def _():
            _wait_one_scatter()

        pltpu.async_copy(h_hbm.at[idx2d.at[j]], gbuf.at[slot], sem_g).wait()
        pltpu.make_async_copy(
            gbuf.at[slot], p_hbm.at[idx2d.at[j]], sem_s).start()
        return 0

    lax.fori_loop(0, n_ch, cbody, 0)

    def dbody(j, _):
        _wait_one_scatter()
        return 0

    lax.fori_loop(0, jnp.minimum(n_ch, 2), dbody, 0)


def _copy_body(h_ref, p_ref, o_ref):
    o_ref[...] = h_ref[...]


def kernel(hidden_states, scores):
    h2 = hidden_states.reshape(R, D)
    s1 = scores.reshape(R)
    mesh = plsc.VectorSubcoreMesh(core_axis_name="c", subcore_axis_name="s")
    route = pl.kernel(
        _route_body,
        out_type=jax.ShapeDtypeStruct((R, D), jnp.float32),
        mesh=mesh,
        compiler_params=pltpu.CompilerParams(needs_layout_passes=False),
        scratch_types=[
            pltpu.VMEM((CW,), jnp.float32),
            pltpu.VMEM((NV, L), jnp.uint32),
            pltpu.VMEM((256,), jnp.int32),
            pltpu.VMEM((WPR, 256), jnp.int32),
            pltpu.VMEM((MAXCH, CHG), jnp.int32),
            pltpu.VMEM((2, CHG, D), jnp.float32),
            pltpu.VMEM_SHARED((NS, 256), jnp.int32),
            pltpu.SemaphoreType.DMA,
            pltpu.SemaphoreType.DMA,
        ],
    )
    p = route(h2, s1)

    out = pl.pallas_call(
        _copy_body,
        grid=(B, T // BT),
        in_specs=[
            pl.BlockSpec((1, BT, D), lambda b, t: (b, t, 0)),
            pl.BlockSpec(memory_space=pl.ANY),
        ],
        out_specs=pl.BlockSpec((1, BT, D), lambda b, t: (b, t, 0)),
        out_shape=jax.ShapeDtypeStruct((B, T, D), hidden_states.dtype),
        input_output_aliases={1: 0},
    )(hidden_states, p.reshape(B, T, D))
    return out


# speedup vs baseline: 1.0272x; 1.0272x over previous
import jax
import jax.numpy as jnp
from jax import lax
from jax.experimental import pallas as pl
from jax.experimental.pallas import tpu as pltpu
from jax.experimental.pallas import tpu_sc as plsc

NC, NS, L = 2, 16, 16
B, T, D = 4, 8192, 1024
K = T // 8
R = B * T
WPR = 8
CW = T // WPR
NV = CW // L
CHG = 32
MAXCH = (CW + CHG - 1) // CHG + 1
BT = 2048


def _scalar(x):
    if getattr(x, "ndim", 0) == 0:
        return x
    return lax.reduce_max(x, (0,))


def _route_body(h_hbm, sc_hbm, p_hbm,
                sbuf, ukeys, hist, merged, idx2d, gbuf, shared, sem_g):
    c = lax.axis_index("c")
    s = lax.axis_index("s")
    b = 2 * c + s // WPR
    jw = s % WPR
    tbase = b * T + jw * CW
    s0 = (s // WPR) * WPR
    lane = lax.iota(jnp.int32, L)

    pltpu.sync_copy(sc_hbm.at[pl.ds(tbase, CW)], sbuf)
    for j in range(NV):
        v = sbuf[pl.ds(L * j, L)]
        xi = lax.bitcast_convert_type(v, jnp.int32)
        sign = xi >> 31
        u = lax.bitcast_convert_type(xi, jnp.uint32) ^ (
            lax.bitcast_convert_type(sign, jnp.uint32) | jnp.uint32(0x80000000))
        ukeys[j] = u

    prefix = jnp.uint32(0)
    krem = jnp.int32(K)
    for p in range(4):
        shift = 24 - 8 * p
        for g in range(256 // L):
            hist[pl.ds(L * g, L)] = jnp.zeros((L,), jnp.int32)

        def hbody(j, _, _p=p, _shift=shift, _prefix=prefix):
            u = ukeys[j]
            binv = lax.bitcast_convert_type(
                (u >> jnp.uint32(_shift)) & jnp.uint32(0xFF), jnp.int32)
            if _p == 0:
                elig = jnp.full((L,), True)
            else:
                elig = (u >> jnp.uint32(_shift + 8)) == _prefix
            counts, lastm = plsc.scan_count(binv, elig)
            plsc.addupdate_scatter(hist, [binv], counts, mask=lastm)
            return 0

        lax.fori_loop(0, NV, hbody, 0)

        pltpu.sync_copy(hist, shared.at[p, s])
        plsc.subcore_barrier()
        pltpu.sync_copy(shared.at[p, pl.ds(s0, WPR)], merged)

        rem = krem
        kin = jnp.int32(1)
        hsel = jnp.zeros((L,), jnp.int32)
        for g in reversed(range(256 // L)):
            hv = jnp.zeros((L,), jnp.int32)
            for w in range(WPR):
                hv = hv + merged[w, pl.ds(L * g, L)]
            tg = lax.reduce_sum(hv, (0,))
            hit = (rem > 0) & (rem <= tg)
            hitv = jnp.full((L,), hit)
            gsel = jnp.where(hit, jnp.int32(g), jnp.int32(0))
            if g == 256 // L - 1:
                g_star = gsel
            else:
                g_star = jnp.where(hit, gsel, g_star)
            kin = jnp.where(hit, rem, kin)
            hsel = jnp.where(hitv, hv, hsel)
            rem = jnp.where(rem > 0, rem - tg, rem)

        rv = lax.rev(hsel, (0,))
        cum = plsc.cumsum(rv)
        fmask = cum >= kin
        f = _scalar(plsc.all_reduce_ffs(fmask))
        cum_f = lax.reduce_sum(jnp.where(lane == f, cum, 0), (0,))
        rv_f = lax.reduce_sum(jnp.where(lane == f, rv, 0), (0,))
        bin_star = (L - 1) - f
        krem = kin - (cum_f - rv_f)
        prefix = (prefix << jnp.uint32(8)) | jnp.uint32(
            g_star * L + bin_star)

    thresh = prefix

    def sbody(j, off):
        u = ukeys[j]
        m = u >= thresh
        mi = jnp.where(m, jnp.int32(1), jnp.int32(0))
        pos = plsc.cumsum(mi) + off - 1
        tok = lane + (tbase + L * j)
        plsc.store_scatter(idx2d, [pos // CHG, pos % CHG], tok, mask=m)
        return off + _scalar(plsc.all_reduce_population_count(m))

    cnt = lax.fori_loop(0, NV, sbody, jnp.int32(0))

    padn = (CHG - lax.rem(cnt, CHG)) % CHG
    for g in range((CHG + L - 1) // L):
        posv = cnt + (g * L) + lane
        m = (g * L + lane) < padn
        plsc.store_scatter(
            idx2d, [posv // CHG, posv % CHG],
            jnp.full((L,), tbase, jnp.int32), mask=m)
    n_ch = (cnt + padn) // CHG

    def cbody(j, _):
        pltpu.async_copy(h_hbm.at[idx2d.at[j]], gbuf, sem_g).wait()
        pltpu.async_copy(gbuf, p_hbm.at[idx2d.at[j]], sem_g).wait()
        return 0

    lax.fori_loop(0, n_ch, cbody, 0)


def _copy_body(h_ref, p_ref, o_ref):
    o_ref[...] = h_ref[...]


def kernel(hidden_states, scores):
    h2 = hidden_states.reshape(R, D)
    s1 = scores.reshape(R)
    mesh = plsc.VectorSubcoreMesh(core_axis_name="c", subcore_axis_name="s")
    route = pl.kernel(
        _route_body,
        out_type=jax.ShapeDtypeStruct((R, D), jnp.float32),
        mesh=mesh,
        compiler_params=pltpu.CompilerParams(needs_layout_passes=False),
        scratch_types=[
            pltpu.VMEM((CW,), jnp.float32),
            pltpu.VMEM((NV, L), jnp.uint32),
            pltpu.VMEM((256,), jnp.int32),
            pltpu.VMEM((WPR, 256), jnp.int32),
            pltpu.VMEM((MAXCH, CHG), jnp.int32),
            pltpu.VMEM((CHG, D), jnp.float32),
            pltpu.VMEM_SHARED((4, NS, 256), jnp.int32),
            pltpu.SemaphoreType.DMA,
        ],
    )
    p = route(h2, s1)

    out = pl.pallas_call(
        _copy_body,
        grid=(B, T // BT),
        in_specs=[
            pl.BlockSpec((1, BT, D), lambda b, t: (b, t, 0)),
            pl.BlockSpec(memory_space=pl.ANY),
        ],
        out_specs=pl.BlockSpec((1, BT, D), lambda b, t: (b, t, 0)),
        out_shape=jax.ShapeDtypeStruct((B, T, D), hidden_states.dtype),
        input_output_aliases={1: 0},
    )(hidden_states, p.reshape(B, T, D))
    return out


# speedup vs baseline: 1.0604x; 1.0323x over previous
import jax
import jax.numpy as jnp
from jax import lax
from jax.experimental import pallas as pl
from jax.experimental.pallas import tpu as pltpu
from jax.experimental.pallas import tpu_sc as plsc

NC, NS, L = 2, 16, 16
B, T, D = 4, 8192, 1024
K = T // 8
R = B * T
WPR = 8
CW = T // WPR
NV = CW // L
CHG = 32
MAXCH = (CW + CHG - 1) // CHG + 1
BT = 2048


def _scalar(x):
    if getattr(x, "ndim", 0) == 0:
        return x
    return lax.reduce_max(x, (0,))


def _route_body(h_hbm, sc_hbm, p_hbm,
                sbuf, ukeys, hist, merged, idx2d, gbuf, shared, sem_g):
    c = lax.axis_index("c")
    s = lax.axis_index("s")
    b = 2 * c + s // WPR
    jw = s % WPR
    tbase = b * T + jw * CW
    s0 = (s // WPR) * WPR
    lane = lax.iota(jnp.int32, L)

    pltpu.sync_copy(sc_hbm.at[pl.ds(tbase, CW)], sbuf)
    for j in range(NV):
        v = sbuf[pl.ds(L * j, L)]
        xi = lax.bitcast_convert_type(v, jnp.int32)
        sign = xi >> 31
        u = lax.bitcast_convert_type(xi, jnp.uint32) ^ (
            lax.bitcast_convert_type(sign, jnp.uint32) | jnp.uint32(0x80000000))
        ukeys[j] = u

    prefix = jnp.uint32(0)
    krem = jnp.int32(K)
    NPASS = 2
    for p in range(NPASS):
        shift = 24 - 8 * p
        for g in range(256 // L):
            hist[pl.ds(L * g, L)] = jnp.zeros((L,), jnp.int32)

        def hbody(j, _, _p=p, _shift=shift, _prefix=prefix):
            u = ukeys[j]
            binv = lax.bitcast_convert_type(
                (u >> jnp.uint32(_shift)) & jnp.uint32(0xFF), jnp.int32)
            if _p == 0:
                elig = jnp.full((L,), True)
            else:
                elig = (u >> jnp.uint32(_shift + 8)) == _prefix
            counts, lastm = plsc.scan_count(binv, elig)
            plsc.addupdate_scatter(hist, [binv], counts, mask=lastm)
            return 0

        lax.fori_loop(0, NV, hbody, 0)

        pltpu.sync_copy(hist, shared.at[p, s])
        plsc.subcore_barrier()
        pltpu.sync_copy(shared.at[p, pl.ds(s0, WPR)], merged)

        rem = krem
        kin = jnp.int32(1)
        hsel = jnp.zeros((L,), jnp.int32)
        for g in reversed(range(256 // L)):
            hv = jnp.zeros((L,), jnp.int32)
            for w in range(WPR):
                hv = hv + merged[w, pl.ds(L * g, L)]
            tg = lax.reduce_sum(hv, (0,))
            hit = (rem > 0) & (rem <= tg)
            hitv = jnp.full((L,), hit)
            gsel = jnp.where(hit, jnp.int32(g), jnp.int32(0))
            if g == 256 // L - 1:
                g_star = gsel
            else:
                g_star = jnp.where(hit, gsel, g_star)
            kin = jnp.where(hit, rem, kin)
            hsel = jnp.where(hitv, hv, hsel)
            rem = jnp.where(rem > 0, rem - tg, rem)

        rv = lax.rev(hsel, (0,))
        cum = plsc.cumsum(rv)
        fmask = cum >= kin
        f = _scalar(plsc.all_reduce_ffs(fmask))
        cum_f = lax.reduce_sum(jnp.where(lane == f, cum, 0), (0,))
        rv_f = lax.reduce_sum(jnp.where(lane == f, rv, 0), (0,))
        bin_star = (L - 1) - f
        krem = kin - (cum_f - rv_f)
        prefix = (prefix << jnp.uint32(8)) | jnp.uint32(
            g_star * L + bin_star)

    thresh = prefix << jnp.uint32(32 - 8 * NPASS)

    def sbody(j, off):
        u = ukeys[j]
        m = u >= thresh
        mi = jnp.where(m, jnp.int32(1), jnp.int32(0))
        pos = plsc.cumsum(mi) + off - 1
        tok = lane + (tbase + L * j)
        plsc.store_scatter(idx2d, [pos // CHG, pos % CHG], tok, mask=m)
        return off + _scalar(plsc.all_reduce_population_count(m))

    cnt = lax.fori_loop(0, NV, sbody, jnp.int32(0))

    padn = (CHG - lax.rem(cnt, CHG)) % CHG
    for g in range((CHG + L - 1) // L):
        posv = cnt + (g * L) + lane
        m = (g * L + lane) < padn
        plsc.store_scatter(
            idx2d, [posv // CHG, posv % CHG],
            jnp.full((L,), tbase, jnp.int32), mask=m)
    n_ch = (cnt + padn) // CHG

    def cbody(j, _):
        pltpu.async_copy(h_hbm.at[idx2d.at[j]], gbuf, sem_g).wait()
        pltpu.async_copy(gbuf, p_hbm.at[idx2d.at[j]], sem_g).wait()
        return 0

    lax.fori_loop(0, n_ch, cbody, 0)


def _copy_body(h_ref, p_ref, o_ref):
    o_ref[...] = h_ref[...]


def kernel(hidden_states, scores):
    h2 = hidden_states.reshape(R, D)
    s1 = scores.reshape(R)
    mesh = plsc.VectorSubcoreMesh(core_axis_name="c", subcore_axis_name="s")
    route = pl.kernel(
        _route_body,
        out_type=jax.ShapeDtypeStruct((R, D), jnp.float32),
        mesh=mesh,
        compiler_params=pltpu.CompilerParams(needs_layout_passes=False),
        scratch_types=[
            pltpu.VMEM((CW,), jnp.float32),
            pltpu.VMEM((NV, L), jnp.uint32),
            pltpu.VMEM((256,), jnp.int32),
            pltpu.VMEM((WPR, 256), jnp.int32),
            pltpu.VMEM((MAXCH, CHG), jnp.int32),
            pltpu.VMEM((CHG, D), jnp.float32),
            pltpu.VMEM_SHARED((4, NS, 256), jnp.int32),
            pltpu.SemaphoreType.DMA,
        ],
    )
    p = route(h2, s1)

    out = pl.pallas_call(
        _copy_body,
        grid=(B, T // BT),
        in_specs=[
            pl.BlockSpec((1, BT, D), lambda b, t: (b, t, 0)),
            pl.BlockSpec(memory_space=pl.ANY),
        ],
        out_specs=pl.BlockSpec((1, BT, D), lambda b, t: (b, t, 0)),
        out_shape=jax.ShapeDtypeStruct((B, T, D), hidden_states.dtype),
        input_output_aliases={1: 0},
    )(hidden_states, p.reshape(B, T, D))
    return out
